# baseline (speedup 1.0000x reference)
import functools

import jax
import jax.numpy as jnp
from jax import lax
from jax.experimental import pallas as pl
from jax.experimental.pallas import tpu as pltpu
from jax.experimental.pallas import tpu_sc as plsc

VOCAB = 100000
N_EMBD = 1024
BLOCK = 2048
BATCH = 4
SEQ = 2048

NC = 2
NS = 16
NW = NC * NS
LANES = 16
B_TOTAL = BATCH * SEQ
B_PER_W = B_TOTAL // NW
C = 32
N_CHUNKS = B_PER_W // C
VPR = N_EMBD // LANES


def _emb_body(x_hbm, wte_hbm, wpe_hbm, out_hbm, idx_v, tok_v, pos_v, gsem):
    wid = lax.axis_index("s") * NC + lax.axis_index("c")
    base = wid * B_PER_W
    s0 = lax.rem(base, BLOCK)

    pltpu.sync_copy(x_hbm.at[pl.ds(base, B_PER_W)], idx_v)

    @pl.loop(0, N_CHUNKS)
    def _chunk(ci):
        off = ci * C
        gather = pltpu.async_copy(
            wte_hbm.at[idx_v.at[pl.ds(off, C)]], tok_v, gsem)
        pltpu.sync_copy(wpe_hbm.at[pl.ds(s0 + off, C)], pos_v)
        gather.wait()

        @pl.loop(0, C)
        def _row(r):
            for k in range(VPR):
                sl = pl.ds(k * LANES, LANES)
                tok_v[r, sl] = tok_v[r, sl] + pos_v[r, sl]

        pltpu.sync_copy(tok_v, out_hbm.at[pl.ds(base + off, C)])


@jax.jit
def kernel(x, wte, wpe):
    xf = x.reshape(-1).astype(jnp.int32)
    mesh = plsc.VectorSubcoreMesh(core_axis_name="c", subcore_axis_name="s")
    run = pl.kernel(
        _emb_body,
        out_type=jax.ShapeDtypeStruct((B_TOTAL, N_EMBD), jnp.float32),
        mesh=mesh,
        scratch_types=[
            pltpu.VMEM((B_PER_W,), jnp.int32),
            pltpu.VMEM((C, N_EMBD), jnp.float32),
            pltpu.VMEM((C, N_EMBD), jnp.float32),
            pltpu.SemaphoreType.DMA,
        ],
    )
    out = run(xf, wte, wpe)
    return out.reshape(BATCH, SEQ, N_EMBD)


# speedup vs baseline: 1.4727x; 1.4727x over previous
import jax
import jax.numpy as jnp
from jax import lax
from jax.experimental import pallas as pl
from jax.experimental.pallas import tpu as pltpu
from jax.experimental.pallas import tpu_sc as plsc

VOCAB = 100000
N_EMBD = 1024
BLOCK = 2048
BATCH = 4
SEQ = 2048

NC = 2
NS = 16
NW = NC * NS
LANES = 16
S_PER_W = SEQ // NW
P = 16
NPC = S_PER_W // P
NSTEP = NPC * BATCH
VPR = N_EMBD // LANES


def _emb_body(x_hbm, wte_hbm, wpe_hbm, out_hbm,
              idx_v, tok, pos, gsem, psem, ssem, isem):
    wid = lax.axis_index("s") * NC + lax.axis_index("c")
    sbase = wid * S_PER_W

    pdesc = {}
    for pc in range(2):
        pdesc[pc] = pltpu.async_copy(
            wpe_hbm.at[pl.ds(sbase + pc * P, P)], pos[pc], psem[pc])

    idesc = [pltpu.async_copy(x_hbm.at[pl.ds(b * SEQ + sbase, S_PER_W)],
                              idx_v.at[pl.ds(b * S_PER_W, S_PER_W)], isem)
             for b in range(BATCH)]
    for d in idesc:
        d.wait()

    def issue_gather(t):
        pc, b = t // BATCH, t % BATCH
        return pltpu.async_copy(
            wte_hbm.at[idx_v.at[pl.ds(b * S_PER_W + pc * P, P)]],
            tok[t % 3], gsem[t % 3])

    gdesc = {0: issue_gather(0), 1: issue_gather(1)}
    sdesc = {}
    for t in range(NSTEP):
        pc, b = t // BATCH, t % BATCH
        gdesc[t].wait()
        if t % BATCH == 0:
            pdesc[pc].wait()

        tb, pb = tok[t % 3], pos[pc % 2]

        @pl.loop(0, P)
        def _row(r):
            for k in range(VPR):
                sl = pl.ds(k * LANES, LANES)
                tb[r, sl] = tb[r, sl] + pb[r, sl]

        sdesc[t] = pltpu.async_copy(
            tb, out_hbm.at[pl.ds(b * SEQ + sbase + pc * P, P)], ssem[t % 3])

        if t % BATCH == BATCH - 1 and pc + 2 < NPC:
            pdesc[pc + 2] = pltpu.async_copy(
                wpe_hbm.at[pl.ds(sbase + (pc + 2) * P, P)],
                pos[pc % 2], psem[pc % 2])

        if t == 0:
            gdesc[2] = issue_gather(2)
        elif t + 2 < NSTEP:
            sdesc[t - 1].wait()
            gdesc[t + 2] = issue_gather(t + 2)

    for t in (NSTEP - 3, NSTEP - 2, NSTEP - 1):
        sdesc[t].wait()


@jax.jit
def kernel(x, wte, wpe):
    xf = x.reshape(-1).astype(jnp.int32)
    mesh = plsc.VectorSubcoreMesh(core_axis_name="c", subcore_axis_name="s")
    run = pl.kernel(
        _emb_body,
        out_type=jax.ShapeDtypeStruct((BATCH * SEQ, N_EMBD), jnp.float32),
        mesh=mesh,
        scratch_types=[
            pltpu.VMEM((BATCH * S_PER_W,), jnp.int32),
            [pltpu.VMEM((P, N_EMBD), jnp.float32) for _ in range(3)],
            [pltpu.VMEM((P, N_EMBD), jnp.float32) for _ in range(2)],
            [pltpu.SemaphoreType.DMA for _ in range(3)],
            [pltpu.SemaphoreType.DMA for _ in range(2)],
            [pltpu.SemaphoreType.DMA for _ in range(3)],
            pltpu.SemaphoreType.DMA,
        ],
    )
    out = run(xf, wte, wpe)
    return out.reshape(BATCH, SEQ, N_EMBD)
